# baseline (speedup 1.0000x reference)
import functools

import jax
import jax.numpy as jnp
from jax import lax
from jax.experimental import pallas as pl
from jax.experimental.pallas import tpu as pltpu
from jax.experimental.pallas import tpu_sc as plsc

NUM_LABELS = 32
PAD = 0
NEG = -1000000000.0
QT = 2048


def _sc_gather_rows(E, qtok, stok):
    info = plsc.get_sparse_core_info()
    NC, NS = info.num_cores, info.num_subcores
    NW = NC * NS
    Q = qtok.shape[0]
    S = stok.shape[0]
    H = E.shape[1]
    q_w = Q // NW
    s_w = S // NW
    n_w = q_w + s_w
    BATCH = 16

    mesh = plsc.VectorSubcoreMesh(core_axis_name="c", subcore_axis_name="s")
    scratch = [
        pltpu.VMEM((n_w,), jnp.int32),
        pltpu.VMEM((n_w, H), jnp.float32),
        pltpu.SemaphoreType.DMA,
        pltpu.SemaphoreType.DMA,
    ]

    @functools.partial(
        pl.kernel,
        mesh=mesh,
        out_type=jax.ShapeDtypeStruct((Q + S, H), jnp.float32),
        scratch_types=scratch,
    )
    def gather_kernel(table_hbm, qtok_hbm, stok_hbm, out_hbm, idx_v, rows_v,
                      sem0, sem1):
        wid = lax.axis_index("s") * NC + lax.axis_index("c")
        qbase = wid * q_w
        sbase = wid * s_w
        pltpu.sync_copy(qtok_hbm.at[pl.ds(qbase, q_w)],
                        idx_v.at[pl.ds(0, q_w)])
        pltpu.sync_copy(stok_hbm.at[pl.ds(sbase, s_w)],
                        idx_v.at[pl.ds(q_w, s_w)])
        sems = (sem0, sem1)
        pending = []
        for b in range(n_w // BATCH):
            sem = sems[b % 2]
            toks = idx_v[pl.ds(b * BATCH, BATCH)]
            batch = []
            for j in range(BATCH):
                i = b * BATCH + j
                batch.append(pltpu.async_copy(
                    table_hbm.at[pl.ds(toks[j], 1)],
                    rows_v.at[pl.ds(i, 1)], sem))
            for h in pending:
                h.wait()
            pending = batch
        for h in pending:
            h.wait()
        pltpu.sync_copy(rows_v.at[pl.ds(0, q_w)],
                        out_hbm.at[pl.ds(qbase, q_w)])
        pltpu.sync_copy(rows_v.at[pl.ds(q_w, s_w)],
                        out_hbm.at[pl.ds(Q + sbase, s_w)])

    return gather_kernel(E, qtok, stok)


def _decode_body(x_ref, sT_ref, lab_ref, qtok_ref, best_ref, near_ref,
                 ynT_s, y2_s):
    @pl.when(pl.program_id(0) == 0)
    def _():
        sT = sT_ref[...]
        ns = jnp.sqrt(jnp.sum(sT * sT, axis=0, keepdims=True))
        ynT = sT / jnp.maximum(ns, 1e-12)
        ynT_s[...] = ynT
        y2_s[...] = jnp.sum(ynT * ynT, axis=0, keepdims=True)

    H = sT_ref.shape[0]
    x = x_ref[:, :H]
    nx = jnp.sqrt(jnp.sum(x * x, axis=1, keepdims=True))
    xn = x / jnp.maximum(nx, 1e-12)
    x2 = jnp.sum(xn * xn, axis=1, keepdims=True)

    d = lax.dot_general(xn, ynT_s[...], (((1,), (0,)), ((), ())),
                        preferred_element_type=jnp.float32)
    scores = 2.0 * d - x2 - y2_s[...]

    lab = lab_ref[...]
    qv = qtok_ref[...] != float(PAD)
    lv = lab != float(PAD)
    scores = jnp.where(jnp.logical_and(qv, lv), scores, NEG)

    m = jnp.max(scores, axis=1, keepdims=True)
    iota = lax.broadcasted_iota(jnp.int32, scores.shape, 1)
    best = jnp.min(jnp.where(scores == m, iota, jnp.int32(2**30)),
                   axis=1, keepdims=True)
    bl = jnp.max(jnp.where(iota == best, lab, 0.0), axis=1, keepdims=True)
    best_ref[...] = bl.astype(jnp.int32)

    sbf = scores.astype(jnp.bfloat16)
    negb = jnp.bfloat16(NEG)
    cols = []
    for l in range(NUM_LABELS):
        sel = jnp.where(lab == float(l), sbf, negb)
        cols.append(jnp.max(sel, axis=1, keepdims=True))
    near = jnp.concatenate(cols, axis=1).astype(jnp.float32)
    near_ref[...] = jnp.where(near < NEG * 0.5, NEG, near)


def _decode(emb, sT, labels_f, qtok_f, interpret=False):
    Q = qtok_f.shape[0]
    W = emb.shape[1]
    H = sT.shape[0]
    S = sT.shape[1]
    grid = (Q // QT,)
    return pl.pallas_call(
        _decode_body,
        grid=grid,
        in_specs=[
            pl.BlockSpec((QT, W), lambda i: (i, 0)),
            pl.BlockSpec((H, S), lambda i: (0, 0)),
            pl.BlockSpec((1, S), lambda i: (0, 0)),
            pl.BlockSpec((QT, 1), lambda i: (i, 0)),
        ],
        out_specs=[
            pl.BlockSpec((QT, 1), lambda i: (i, 0)),
            pl.BlockSpec((QT, NUM_LABELS), lambda i: (i, 0)),
        ],
        out_shape=[
            jax.ShapeDtypeStruct((Q, 1), jnp.int32),
            jax.ShapeDtypeStruct((Q, NUM_LABELS), jnp.float32),
        ],
        scratch_shapes=[
            pltpu.VMEM((H, S), jnp.float32),
            pltpu.VMEM((1, S), jnp.float32),
        ],
        interpret=interpret,
    )(emb, sT, labels_f, qtok_f)


def kernel(support, label_support, query, E):
    support = support.astype(jnp.int32)
    qflat = query.astype(jnp.int32).reshape(-1)
    S = support.shape[0]
    H = E.shape[1]
    Q = qflat.shape[0]

    emb = _sc_gather_rows(E, qflat, support)

    sT = emb[Q:, :H].T
    labels_f = label_support.astype(jnp.float32).reshape(1, S)
    qtok_f = qflat.astype(jnp.float32).reshape(-1, 1)

    best, near = _decode(emb, sT, labels_f, qtok_f)
    return (best.reshape(query.shape),
            near.reshape(query.shape + (NUM_LABELS,)))


# speedup vs baseline: 1.1669x; 1.1669x over previous
import functools

import jax
import jax.numpy as jnp
from jax import lax
from jax.experimental import pallas as pl
from jax.experimental.pallas import tpu as pltpu
from jax.experimental.pallas import tpu_sc as plsc

NUM_LABELS = 32
PAD = 0
NEG = -1000000000.0
QT = 2048


def _sc_gather_rows(E, qtok, stok):
    info = plsc.get_sparse_core_info()
    NC, NS = info.num_cores, info.num_subcores
    NW = NC * NS
    Q = qtok.shape[0]
    S = stok.shape[0]
    H = E.shape[1]
    q_w = Q // NW
    s_w = S // NW
    n_w = q_w + s_w
    BATCH = 16

    mesh = plsc.VectorSubcoreMesh(core_axis_name="c", subcore_axis_name="s")
    scratch = [
        pltpu.VMEM((n_w,), jnp.int32),
        pltpu.VMEM((n_w, H), jnp.float32),
        pltpu.SemaphoreType.DMA,
        pltpu.SemaphoreType.DMA,
    ]

    @functools.partial(
        pl.kernel,
        mesh=mesh,
        out_type=jax.ShapeDtypeStruct((Q + S, H), jnp.float32),
        scratch_types=scratch,
    )
    def gather_kernel(table_hbm, qtok_hbm, stok_hbm, out_hbm, idx_v, rows_v,
                      sem0, sem1):
        wid = lax.axis_index("s") * NC + lax.axis_index("c")
        qbase = wid * q_w
        sbase = wid * s_w
        pltpu.sync_copy(qtok_hbm.at[pl.ds(qbase, q_w)],
                        idx_v.at[pl.ds(0, q_w)])
        pltpu.sync_copy(stok_hbm.at[pl.ds(sbase, s_w)],
                        idx_v.at[pl.ds(q_w, s_w)])
        sems = (sem0, sem1)
        pending = []
        for b in range(n_w // BATCH):
            sem = sems[b % 2]
            toks = idx_v[pl.ds(b * BATCH, BATCH)]
            batch = []
            for j in range(BATCH):
                i = b * BATCH + j
                batch.append(pltpu.async_copy(
                    table_hbm.at[pl.ds(toks[j], 1)],
                    rows_v.at[pl.ds(i, 1)], sem))
            for h in pending:
                h.wait()
            pending = batch
        for h in pending:
            h.wait()
        pltpu.sync_copy(rows_v.at[pl.ds(0, q_w)],
                        out_hbm.at[pl.ds(qbase, q_w)])
        pltpu.sync_copy(rows_v.at[pl.ds(q_w, s_w)],
                        out_hbm.at[pl.ds(Q + sbase, s_w)])

    return gather_kernel(E, qtok, stok)


WIN = 384


def _window_starts(S):
    per = S // NUM_LABELS
    return [min(max(((per * l - 96) // 128) * 128, 0), S - WIN)
            for l in range(NUM_LABELS)]


def _decode_body(flag_ref, x_ref, sT_ref, lab_ref, qtok_ref, perm_ref,
                 best_ref, near_ref, ynT_s, y2_s):
    @pl.when(pl.program_id(0) == 0)
    def _():
        sT = sT_ref[...]
        ns = jnp.sqrt(jnp.sum(sT * sT, axis=0, keepdims=True))
        ynT = sT / jnp.maximum(ns, 1e-12)
        ynT_s[...] = ynT
        y2_s[...] = jnp.sum(ynT * ynT, axis=0, keepdims=True)

    H = sT_ref.shape[0]
    x = x_ref[:, :H]
    nx = jnp.sqrt(jnp.sum(x * x, axis=1, keepdims=True))
    xn = x / jnp.maximum(nx, 1e-12)
    x2 = jnp.sum(xn * xn, axis=1, keepdims=True)

    d = lax.dot_general(xn, ynT_s[...], (((1,), (0,)), ((), ())),
                        preferred_element_type=jnp.float32)
    scores = 2.0 * d - x2 - y2_s[...]

    lab = lab_ref[...]
    qv = qtok_ref[...] != float(PAD)
    lv = lab != float(PAD)
    scores = jnp.where(jnp.logical_and(qv, lv), scores, NEG)

    perm = perm_ref[...]
    m = jnp.max(scores, axis=1, keepdims=True)
    best = jnp.min(jnp.where(scores == m, perm, jnp.float32(2**30)),
                   axis=1, keepdims=True)
    bl = jnp.max(jnp.where(perm == best, lab, 0.0), axis=1, keepdims=True)
    best_ref[...] = bl.astype(jnp.int32)

    S = scores.shape[1]
    sbf = scores.astype(jnp.bfloat16)
    negb = jnp.bfloat16(NEG)
    starts = _window_starts(S)
    cols = []
    for l in range(NUM_LABELS):
        a = starts[l]
        sel = jnp.where(lab[:, a:a + WIN] == float(l),
                        sbf[:, a:a + WIN], negb)
        cols.append(jnp.max(sel, axis=1, keepdims=True))
    near = jnp.concatenate(cols, axis=1).astype(jnp.float32)
    near_ref[...] = jnp.where(near < NEG * 0.5, NEG, near)

    @pl.when(flag_ref[0] != 0)
    def _():
        cols_f = []
        for l in range(NUM_LABELS):
            sel = jnp.where(lab == float(l), sbf, negb)
            cols_f.append(jnp.max(sel, axis=1, keepdims=True))
        near_f = jnp.concatenate(cols_f, axis=1).astype(jnp.float32)
        near_ref[...] = jnp.where(near_f < NEG * 0.5, NEG, near_f)


def _decode(flag, emb, sT, labels_f, qtok_f, perm_f, interpret=False):
    Q = qtok_f.shape[0]
    W = emb.shape[1]
    H = sT.shape[0]
    S = sT.shape[1]
    grid = (Q // QT,)
    grid_spec = pltpu.PrefetchScalarGridSpec(
        num_scalar_prefetch=1,
        grid=grid,
        in_specs=[
            pl.BlockSpec((QT, W), lambda i, f: (i, 0)),
            pl.BlockSpec((H, S), lambda i, f: (0, 0)),
            pl.BlockSpec((1, S), lambda i, f: (0, 0)),
            pl.BlockSpec((QT, 1), lambda i, f: (i, 0)),
            pl.BlockSpec((1, S), lambda i, f: (0, 0)),
        ],
        out_specs=[
            pl.BlockSpec((QT, 1), lambda i, f: (i, 0)),
            pl.BlockSpec((QT, NUM_LABELS), lambda i, f: (i, 0)),
        ],
        scratch_shapes=[
            pltpu.VMEM((H, S), jnp.float32),
            pltpu.VMEM((1, S), jnp.float32),
        ],
    )
    return pl.pallas_call(
        _decode_body,
        grid_spec=grid_spec,
        out_shape=[
            jax.ShapeDtypeStruct((Q, 1), jnp.int32),
            jax.ShapeDtypeStruct((Q, NUM_LABELS), jnp.float32),
        ],
        interpret=interpret,
    )(flag, emb, sT, labels_f, qtok_f, perm_f)


def kernel(support, label_support, query, E):
    support = support.astype(jnp.int32)
    qflat = query.astype(jnp.int32).reshape(-1)
    S = support.shape[0]
    H = E.shape[1]
    Q = qflat.shape[0]

    labels_i = label_support.astype(jnp.int32)
    perm = jnp.argsort(labels_i)
    stok = support[perm]
    lab_sorted = labels_i[perm]
    starts = jnp.asarray(_window_starts(S), jnp.int32)
    a_of_col = jnp.take(starts, lab_sorted)
    col = jnp.arange(S, dtype=jnp.int32)
    escaped = (col < a_of_col) | (col >= a_of_col + WIN)
    flag = jnp.any(escaped).astype(jnp.int32).reshape(1)

    emb = _sc_gather_rows(E, qflat, stok)

    sT = emb[Q:, :H].T
    labels_f = lab_sorted.astype(jnp.float32).reshape(1, S)
    perm_f = perm.astype(jnp.float32).reshape(1, S)
    qtok_f = qflat.astype(jnp.float32).reshape(-1, 1)

    best, near = _decode(flag, emb, sT, labels_f, qtok_f, perm_f)
    return (best.reshape(query.shape),
            near.reshape(query.shape + (NUM_LABELS,)))
